# baseline (speedup 1.0000x reference)
import functools

import jax
import jax.numpy as jnp
from jax import lax
from jax.experimental import pallas as pl
from jax.experimental.pallas import tpu as pltpu
from jax.experimental.pallas import tpu_sc as plsc

_L = 16


def _make_emb_kernel(B, V, D, NC, NS):
    NW = NC * NS
    b_per_w = B // NW
    n_grp = b_per_w // _L
    V_main = V // 128 * 128
    n_tail = V - V_main

    mesh = plsc.VectorSubcoreMesh(core_axis_name="c", subcore_axis_name="s")

    @functools.partial(
        pl.kernel,
        out_type=jax.ShapeDtypeStruct((4, 8, B), jnp.float32),
        mesh=mesh,
        scratch_types=[
            pltpu.VMEM((b_per_w,), jnp.int32),
            pltpu.VMEM((b_per_w,), jnp.int32),
            pltpu.VMEM((D, b_per_w), jnp.float32),
            pltpu.VMEM((D, 128), jnp.float32),
            pltpu.SemaphoreType.DMA,
            pltpu.SemaphoreType.DMA,
        ],
        compiler_params=pltpu.CompilerParams(
            needs_layout_passes=False, use_tc_tiling_on_sc=False
        ),
    )
    def emb_kernel(idx_hbm, tab3, tail3, out3, idx_v, clp_v, gat_v, tail_v, sem, osem):
        wid = lax.axis_index("s") * NC + lax.axis_index("c")
        pltpu.sync_copy(idx_hbm.at[wid], idx_v)
        for t in range(4):
            for s in range(8):
                pltpu.sync_copy(tail3.at[t, s], tail_v.at[t * 8 + s])
        for g in range(n_grp):
            iv = idx_v[pl.ds(g * _L, _L)]
            clp_v[pl.ds(g * _L, _L)] = jnp.minimum(iv, V_main - 1)
        gathers = []
        for t in range(4):
            for s in range(8):
                gathers.append(
                    pltpu.async_copy(
                        tab3.at[t, s, pl.ds(0, V_main)].at[clp_v],
                        gat_v.at[t * 8 + s],
                        sem,
                    )
                )
        for g in gathers:
            g.wait()
        def fix_group(g, _):
            iv = idx_v[pl.ds(g * _L, _L)]
            m = iv >= V_main
            nt = plsc.all_reduce_population_count(m)

            @pl.when(nt[0] > 0)
            def _():
                loc = jnp.minimum(iv - V_main, n_tail - 1)
                for d in range(D):
                    vals = plsc.load_gather(
                        tail_v, [jnp.full((_L,), d, jnp.int32), loc]
                    )
                    cur = gat_v[d, pl.ds(g * _L, _L)]
                    gat_v[d, pl.ds(g * _L, _L)] = jnp.where(m, vals, cur)
            return ()

        lax.fori_loop(0, n_grp, fix_group, (), unroll=False)
        writes = []
        for t in range(4):
            for s in range(8):
                writes.append(
                    pltpu.async_copy(
                        gat_v.at[t * 8 + s],
                        out3.at[t, s, pl.ds(wid * b_per_w, b_per_w)],
                        osem,
                    )
                )
        for w in writes:
            w.wait()

    return emb_kernel


def kernel(cls, cls_emb):
    (B,) = cls.shape
    V, D = cls_emb.shape
    info = plsc.get_sparse_core_info()
    NC, NS = info.num_cores, info.num_subcores
    NW = NC * NS
    V_main = V // 128 * 128
    idx = cls.astype(jnp.int32).reshape(NW, B // NW)
    tab3 = cls_emb.T.reshape(4, 8, V)
    tail = jnp.zeros((128, D), jnp.float32).at[: V - V_main].set(cls_emb[V_main:])
    tail3 = tail.T.reshape(4, 8, 128)
    out3 = _make_emb_kernel(B, V, D, NC, NS)(idx, tab3, tail3)
    return out3.reshape(D, B).T


# speedup vs baseline: 1.0031x; 1.0031x over previous
import functools

import jax
import jax.numpy as jnp
from jax import lax
from jax.experimental import pallas as pl
from jax.experimental.pallas import tpu as pltpu
from jax.experimental.pallas import tpu_sc as plsc

_L = 16


def _make_emb_kernel(B, V, D, NC, NS):
    NW = NC * NS
    b_per_w = B // NW
    n_grp = b_per_w // _L
    n_slab = D

    mesh = plsc.VectorSubcoreMesh(core_axis_name="c", subcore_axis_name="s")

    @functools.partial(
        pl.kernel,
        out_type=jax.ShapeDtypeStruct((4, 8, B), jnp.float32),
        mesh=mesh,
        scratch_types=[
            pltpu.VMEM((b_per_w,), jnp.int32),
            pltpu.VMEM((b_per_w,), jnp.int32),
            pltpu.VMEM((b_per_w,), jnp.int32),
            pltpu.VMEM((b_per_w, _L), jnp.float32),
            pltpu.VMEM((b_per_w, _L), jnp.float32),
            pltpu.VMEM((D, b_per_w), jnp.float32),
            pltpu.SemaphoreType.DMA,
            pltpu.SemaphoreType.DMA,
            pltpu.SemaphoreType.DMA,
        ],
        compiler_params=pltpu.CompilerParams(
            needs_layout_passes=False, use_tc_tiling_on_sc=False
        ),
    )
    def emb_kernel(
        idx_hbm, tab4, out3, idx_v, cid_v, sub_v, buf0, buf1, gat_v, sem0, sem1, osem
    ):
        wid = lax.axis_index("s") * NC + lax.axis_index("c")
        pltpu.sync_copy(idx_hbm.at[wid], idx_v)
        for g in range(n_grp):
            iv = idx_v[pl.ds(g * _L, _L)]
            cid_v[pl.ds(g * _L, _L)] = lax.shift_right_logical(iv, 4)
            sub_v[pl.ds(g * _L, _L)] = lax.bitwise_and(iv, 15)

        bufs = (buf0, buf1)
        sems = (sem0, sem1)
        descr = [None] * n_slab
        descr[0] = pltpu.async_copy(tab4.at[0, 0].at[cid_v], buf0, sem0)
        writes = []
        for d in range(n_slab):
            if d + 1 < n_slab:
                t, s = (d + 1) // 8, (d + 1) % 8
                descr[d + 1] = pltpu.async_copy(
                    tab4.at[t, s].at[cid_v], bufs[(d + 1) % 2], sems[(d + 1) % 2]
                )
            descr[d].wait()
            src = bufs[d % 2]
            for g in range(n_grp):
                rvec = lax.iota(jnp.int32, _L) + g * _L
                vals = plsc.load_gather(src, [rvec, sub_v[pl.ds(g * _L, _L)]])
                gat_v[d, pl.ds(g * _L, _L)] = vals
            writes.append(
                pltpu.async_copy(
                    gat_v.at[d],
                    out3.at[d // 8, d % 8, pl.ds(wid * b_per_w, b_per_w)],
                    osem,
                )
            )
        for w in writes:
            w.wait()

    return emb_kernel


def kernel(cls, cls_emb):
    (B,) = cls.shape
    V, D = cls_emb.shape
    info = plsc.get_sparse_core_info()
    NC, NS = info.num_cores, info.num_subcores
    NW = NC * NS
    idx = cls.astype(jnp.int32).reshape(NW, B // NW)
    tab4 = cls_emb.T.reshape(4, 8, V // _L, _L)
    out3 = _make_emb_kernel(B, V, D, NC, NS)(idx, tab4)
    return out3.reshape(D, B).T


# speedup vs baseline: 1.0033x; 1.0002x over previous
import functools

import jax
import jax.numpy as jnp
from jax import lax
from jax.experimental import pallas as pl
from jax.experimental.pallas import tpu as pltpu
from jax.experimental.pallas import tpu_sc as plsc

_L = 16


def _make_emb_kernel(B, V, D, NC, NS):
    NW = NC * NS
    b_per_w = B // NW
    n_grp = b_per_w // _L

    mesh = plsc.VectorSubcoreMesh(core_axis_name="c", subcore_axis_name="s")

    @functools.partial(
        pl.kernel,
        out_type=jax.ShapeDtypeStruct((4, 8, B), jnp.float32),
        mesh=mesh,
        scratch_types=[
            pltpu.VMEM((b_per_w,), jnp.int32),
            pltpu.VMEM((b_per_w,), jnp.int32),
            pltpu.VMEM((b_per_w,), jnp.int32),
            pltpu.VMEM((2, D * _L, _L), jnp.float32),
            pltpu.VMEM((D, b_per_w), jnp.float32),
            pltpu.SemaphoreType.DMA,
            pltpu.SemaphoreType.DMA,
            pltpu.SemaphoreType.DMA,
        ],
        compiler_params=pltpu.CompilerParams(
            needs_layout_passes=False, use_tc_tiling_on_sc=False
        ),
    )
    def emb_kernel(
        idx_hbm, tab4, out3, idx_v, cid_v, sub_v, buf, gat_v, sem0, sem1, osem
    ):
        wid = lax.axis_index("s") * NC + lax.axis_index("c")
        pltpu.sync_copy(idx_hbm.at[wid], idx_v)
        for g in range(n_grp):
            iv = idx_v[pl.ds(g * _L, _L)]
            cid_v[pl.ds(g * _L, _L)] = lax.shift_right_logical(iv, 4)
            sub_v[pl.ds(g * _L, _L)] = lax.bitwise_and(iv, 15)

        sems = (sem0, sem1)

        def fire(g, par):
            cvec = cid_v[pl.ds(g * _L, _L)]
            for d in range(D):
                pltpu.async_copy(
                    tab4.at[d // 8, d % 8].at[cvec],
                    buf.at[par, pl.ds(d * _L, _L)],
                    sems[par],
                )

        def drain_extract(g, par):
            pltpu.make_async_copy(
                tab4.at[0, 0, pl.ds(0, D * _L)],
                buf.at[par],
                sems[par],
            ).wait()
            sub = sub_v[pl.ds(g * _L, _L)]
            rvec = lax.iota(jnp.int32, _L)
            for d in range(D):
                vals = plsc.load_gather(
                    buf, [jnp.full((_L,), par, jnp.int32), rvec + d * _L, sub]
                )
                gat_v[d, pl.ds(g * _L, _L)] = vals

        fire(0, 0)

        def body(g, _):
            par = lax.rem(g, 2)

            @pl.when(par == 0)
            def _():
                fire(g + 1, 1)
                drain_extract(g, 0)

            @pl.when(par == 1)
            def _():
                fire(g + 1, 0)
                drain_extract(g, 1)
            return ()

        lax.fori_loop(0, n_grp - 1, body, (), unroll=False)
        drain_extract(n_grp - 1, (n_grp - 1) % 2)

        writes = []
        for d in range(D):
            writes.append(
                pltpu.async_copy(
                    gat_v.at[d],
                    out3.at[d // 8, d % 8, pl.ds(wid * b_per_w, b_per_w)],
                    osem,
                )
            )
        for w in writes:
            w.wait()

    return emb_kernel


def kernel(cls, cls_emb):
    (B,) = cls.shape
    V, D = cls_emb.shape
    info = plsc.get_sparse_core_info()
    NC, NS = info.num_cores, info.num_subcores
    NW = NC * NS
    idx = cls.astype(jnp.int32).reshape(NW, B // NW)
    tab4 = cls_emb.T.reshape(4, 8, V // _L, _L)
    out3 = _make_emb_kernel(B, V, D, NC, NS)(idx, tab4)
    return out3.reshape(D, B).T


# speedup vs baseline: 1.0055x; 1.0022x over previous
import functools

import jax
import jax.numpy as jnp
from jax import lax
from jax.experimental import pallas as pl
from jax.experimental.pallas import tpu as pltpu
from jax.experimental.pallas import tpu_sc as plsc

_L = 16


def _make_emb_kernel(B, V, D, NC, NS):
    NW = NC * NS
    b_per_w = B // NW
    n_grp = b_per_w // _L

    mesh = plsc.VectorSubcoreMesh(core_axis_name="c", subcore_axis_name="s")

    @functools.partial(
        pl.kernel,
        out_type=jax.ShapeDtypeStruct((4, 8, B), jnp.float32),
        mesh=mesh,
        scratch_types=[
            pltpu.VMEM((b_per_w,), jnp.int32),
            pltpu.VMEM((b_per_w,), jnp.int32),
            pltpu.VMEM((b_per_w,), jnp.int32),
            pltpu.VMEM((2, D * _L, _L), jnp.float32),
            pltpu.VMEM((D, b_per_w), jnp.float32),
            pltpu.SemaphoreType.DMA,
            pltpu.SemaphoreType.DMA,
            pltpu.SemaphoreType.DMA,
        ],
        compiler_params=pltpu.CompilerParams(
            needs_layout_passes=False, use_tc_tiling_on_sc=False
        ),
    )
    def emb_kernel(
        idx_hbm, tab4, out3, idx_v, cid_v, sub_v, buf, gat_v, sem0, sem1, osem
    ):
        wid = lax.axis_index("s") * NC + lax.axis_index("c")
        pltpu.sync_copy(idx_hbm.at[wid], idx_v)
        for g in range(n_grp):
            iv = idx_v[pl.ds(g * _L, _L)]
            cid_v[pl.ds(g * _L, _L)] = lax.shift_right_logical(iv, 4)
            sub_v[pl.ds(g * _L, _L)] = lax.bitwise_and(iv, 15)

        sems = (sem0, sem1)

        def fire(g, par):
            cvec = cid_v[pl.ds(g * _L, _L)]
            idx = plsc.Indices(cvec, ignored_value=2**31 - 1)
            for d in range(D):
                pltpu.async_copy(
                    tab4.at[d // 8, d % 8].at[idx],
                    buf.at[par, pl.ds(d * _L, _L)],
                    sems[par],
                )

        def drain_extract(g, par):
            pltpu.make_async_copy(
                tab4.at[0, 0, pl.ds(0, D * _L)],
                buf.at[par],
                sems[par],
            ).wait()
            sub = sub_v[pl.ds(g * _L, _L)]
            rvec = lax.iota(jnp.int32, _L)
            for d in range(D):
                vals = plsc.load_gather(
                    buf, [jnp.full((_L,), par, jnp.int32), rvec + d * _L, sub]
                )
                gat_v[d, pl.ds(g * _L, _L)] = vals

        fire(0, 0)

        def body(g, _):
            par = lax.rem(g, 2)

            @pl.when(par == 0)
            def _():
                fire(g + 1, 1)
                drain_extract(g, 0)

            @pl.when(par == 1)
            def _():
                fire(g + 1, 0)
                drain_extract(g, 1)
            return ()

        lax.fori_loop(0, n_grp - 1, body, (), unroll=False)
        drain_extract(n_grp - 1, (n_grp - 1) % 2)

        writes = []
        for d in range(D):
            writes.append(
                pltpu.async_copy(
                    gat_v.at[d],
                    out3.at[d // 8, d % 8, pl.ds(wid * b_per_w, b_per_w)],
                    osem,
                )
            )
        for w in writes:
            w.wait()

    return emb_kernel


def kernel(cls, cls_emb):
    (B,) = cls.shape
    V, D = cls_emb.shape
    info = plsc.get_sparse_core_info()
    NC, NS = info.num_cores, info.num_subcores
    NW = NC * NS
    idx = cls.astype(jnp.int32).reshape(NW, B // NW)
    tab4 = cls_emb.T.reshape(4, 8, V // _L, _L)
    out3 = _make_emb_kernel(B, V, D, NC, NS)(idx, tab4)
    return out3.reshape(D, B).T


# speedup vs baseline: 1.0090x; 1.0036x over previous
import functools

import jax
import jax.numpy as jnp
from jax import lax
from jax.experimental import pallas as pl
from jax.experimental.pallas import tpu as pltpu
from jax.experimental.pallas import tpu_sc as plsc

_L = 16


def _make_emb_kernel(B, V, D, NC, NS):
    NW = NC * NS
    b_per_w = B // NW
    n_grp = b_per_w // _L

    mesh = plsc.VectorSubcoreMesh(core_axis_name="c", subcore_axis_name="s")

    @functools.partial(
        pl.kernel,
        out_type=jax.ShapeDtypeStruct((4, 8, B), jnp.float32),
        mesh=mesh,
        scratch_types=[
            pltpu.VMEM((b_per_w,), jnp.int32),
            pltpu.VMEM((D, b_per_w), jnp.float32),
            pltpu.SemaphoreType.DMA,
            pltpu.SemaphoreType.DMA,
        ],
        compiler_params=pltpu.CompilerParams(
            needs_layout_passes=False, use_tc_tiling_on_sc=False
        ),
    )
    def emb_kernel(idx_hbm, tab1, out3, idx_v, gat_v, sem, osem):
        wid = lax.axis_index("s") * NC + lax.axis_index("c")
        pltpu.sync_copy(idx_hbm.at[wid], idx_v)

        def fire(g, _):
            iv = idx_v[pl.ds(g * _L, _L)]
            for d in range(D):
                off = iv + jnp.int32(d * V)
                pltpu.async_copy(
                    tab1.at[off],
                    gat_v.at[d, pl.ds(g * _L, _L)],
                    sem,
                )
            return ()

        lax.fori_loop(0, n_grp, fire, (), unroll=False)
        for t in range(4):
            pltpu.make_async_copy(
                out3.at[t, :, pl.ds(0, b_per_w)],
                gat_v.at[pl.ds(t * 8, 8)],
                sem,
            ).wait()
        writes = []
        for d in range(D):
            writes.append(
                pltpu.async_copy(
                    gat_v.at[d],
                    out3.at[d // 8, d % 8, pl.ds(wid * b_per_w, b_per_w)],
                    osem,
                )
            )
        for w in writes:
            w.wait()

    return emb_kernel


def kernel(cls, cls_emb):
    (B,) = cls.shape
    V, D = cls_emb.shape
    info = plsc.get_sparse_core_info()
    NC, NS = info.num_cores, info.num_subcores
    NW = NC * NS
    idx = cls.astype(jnp.int32).reshape(NW, B // NW)
    tab1 = cls_emb.T.reshape(-1)
    out3 = _make_emb_kernel(B, V, D, NC, NS)(idx, tab1)
    return out3.reshape(D, B).T
